# baseline (speedup 1.0000x reference)
import functools

import jax
import jax.numpy as jnp
from jax import lax
from jax.experimental import pallas as pl
from jax.experimental.pallas import tpu as pltpu
from jax.experimental.pallas import tpu_sc as plsc

_NUM_WORKERS = 32
_CHUNK = 128
_LANES = 16
_U = 64


def _flatten_body(in_ref, out_ref):
    for t in range(_U):
        out_ref[pl.ds(t * 1024, 1024)] = in_ref[:, t * 128:(t + 1) * 128].reshape(1024)


@functools.lru_cache(maxsize=None)
def _make_flatten(V, D):
    n_cb = (D + 7) // 8
    n_bl = (V + 128 * _U - 1) // (128 * _U)
    out_words = n_cb * n_bl * _U * 1024

    def call(tbl):
        return pl.pallas_call(
            _flatten_body,
            grid=(n_cb, n_bl),
            in_specs=[pl.BlockSpec((8, 128 * _U), lambda cb, b: (cb, b))],
            out_specs=pl.BlockSpec((_U * 1024,), lambda cb, b: (cb * n_bl + b,)),
            out_shape=jax.ShapeDtypeStruct((out_words,), jnp.float32),
        )(tbl)

    return call, n_bl * _U


@functools.lru_cache(maxsize=None)
def _make_sc_gather(V, D, B, tiles_per_cb, n_granules):
    b_per_w = B // _NUM_WORKERS
    n_ch = b_per_w // _CHUNK
    n_grp = _CHUNK // _LANES
    mesh = plsc.VectorSubcoreMesh(core_axis_name="c", subcore_axis_name="s")

    @functools.partial(
        pl.kernel,
        mesh=mesh,
        out_type=jax.ShapeDtypeStruct((D, B), jnp.float32),
        scratch_types=(
            [pltpu.VMEM((b_per_w,), jnp.int32)]
            + [pltpu.VMEM((_CHUNK,), jnp.int32)
               for _ in range(n_ch * D)]
            + [pltpu.VMEM((_CHUNK, _LANES), jnp.float32)
               for _ in range(n_ch * D)]
            + [pltpu.VMEM((D, b_per_w), jnp.float32)]
            + [pltpu.SemaphoreType.DMA for _ in range(n_ch)]
        ),
        compiler_params=pltpu.CompilerParams(use_tc_tiling_on_sc=False, needs_layout_passes=False),
    )
    def sc_gather(table_hbm, idx_hbm, out_hbm, *scratch):
        idx_v = scratch[0]
        g_bufs = scratch[1:1 + n_ch * D]
        d_bufs = scratch[1 + n_ch * D:1 + 2 * n_ch * D]
        out_v = scratch[1 + 2 * n_ch * D]
        sems = scratch[2 + 2 * n_ch * D:2 + 2 * n_ch * D + n_ch]

        wid = lax.axis_index("s") * 2 + lax.axis_index("c")
        pltpu.sync_copy(idx_hbm.at[wid], idx_v)

        iota = lax.iota(jnp.int32, _LANES)
        copies = []
        for j in range(n_ch):
            for k in range(n_grp):
                ids = idx_v[pl.ds(j * _CHUNK + k * _LANES, _LANES)]
                gbase = (lax.shift_right_logical(ids, 7) * 64
                         + jnp.bitwise_and(lax.shift_right_logical(ids, 4), 7))
                for d in range(D):
                    cb, s = d >> 3, d & 7
                    g_bufs[j * D + d][pl.ds(k * _LANES, _LANES)] = (
                        gbase + (cb * tiles_per_cb * 64 + s * 8))
            for d in range(D):
                copies.append(pltpu.async_copy(
                    table_hbm.at[g_bufs[j * D + d]], d_bufs[j * D + d], sems[j]))

        for j in range(n_ch):
            for d in range(D):
                copies[j * D + d].wait()
            for k in range(n_grp):
                ids = idx_v[pl.ds(j * _CHUNK + k * _LANES, _LANES)]
                o = jnp.bitwise_and(ids, 15)
                rows = iota + (k * _LANES)
                outcol = rows + (j * _CHUNK)
                for d in range(D):
                    v = plsc.load_gather(d_bufs[j * D + d], [rows, o])
                    plsc.store_scatter(
                        out_v, [jnp.full((_LANES,), d, jnp.int32), outcol], v)

        for d in range(D):
            pltpu.sync_copy(out_v.at[d], out_hbm.at[d, pl.ds(wid * b_per_w, b_per_w)])

    return sc_gather


def _tc_math(c_ref, d_ref, o_ref):
    C = [c_ref[k] for k in range(16)]
    d = [d_ref[k] for k in range(9)]
    a1x, a1y, a1z = d[3] + 1.0, d[4], d[5]
    a2x, a2y, a2z = d[6], d[7] + 1.0, d[8]
    n1 = lax.rsqrt(a1x * a1x + a1y * a1y + a1z * a1z)
    b1x, b1y, b1z = a1x * n1, a1y * n1, a1z * n1
    dot = b1x * a2x + b1y * a2y + b1z * a2z
    u2x, u2y, u2z = a2x - dot * b1x, a2y - dot * b1y, a2z - dot * b1z
    n2 = lax.rsqrt(u2x * u2x + u2y * u2y + u2z * u2z)
    b2x, b2y, b2z = u2x * n2, u2y * n2, u2z * n2
    b3x = b1y * b2z - b1z * b2y
    b3y = b1z * b2x - b1x * b2z
    b3z = b1x * b2y - b1y * b2x
    rot = ((b1x, b1y, b1z), (b2x, b2y, b2z), (b3x, b3y, b3z))
    dx = (d[0], d[1], d[2])
    for r in range(4):
        Cr = [C[4 * r + k] for k in range(4)]
        for cc in range(3):
            o_ref[4 * r + cc] = (
                Cr[0] * rot[0][cc] + Cr[1] * rot[1][cc] + Cr[2] * rot[2][cc]
            )
        o_ref[4 * r + 3] = Cr[0] * dx[0] + Cr[1] * dx[1] + Cr[2] * dx[2] + Cr[3]


def kernel(camtoworlds, embed_ids, embeds):
    B = camtoworlds.shape[0]
    V, D = embeds.shape
    R = B // 128
    flatten, tiles_per_cb = _make_flatten(V, D)
    flat = flatten(embeds.T)
    table = flat.reshape(flat.shape[0] // _LANES, _LANES)
    ids = embed_ids.astype(jnp.int32).reshape(_NUM_WORKERS, B // _NUM_WORKERS)
    return flat[:B * 16].reshape(B, 4, 4)
    dT = _make_sc_gather(V, D, B, tiles_per_cb, table.shape[0])(
        table, ids).reshape(D, R, 128)
    cT = camtoworlds.reshape(B, 16).T.reshape(16, R, 128)
    oT = pl.pallas_call(
        _tc_math,
        out_shape=jax.ShapeDtypeStruct((16, R, 128), jnp.float32),
    )(cT, dT)
    return oT.reshape(16, B).T.reshape(B, 4, 4)


# speedup vs baseline: 1.2094x; 1.2094x over previous
import functools

import jax
import jax.numpy as jnp
from jax import lax
from jax.experimental import pallas as pl
from jax.experimental.pallas import tpu as pltpu
from jax.experimental.pallas import tpu_sc as plsc

_NUM_WORKERS = 32
_CHUNK = 128
_LANES = 16
_U = 64


def _flatten_body(in_ref, out_ref):
    for t in range(_U):
        out_ref[t] = in_ref[:, pl.ds(t * 128, 128)]


@functools.lru_cache(maxsize=None)
def _make_flatten(V, D):
    n_cb = (D + 7) // 8
    n_bl = (V + 128 * _U - 1) // (128 * _U)
    n_tiles = n_cb * n_bl * _U

    def call(tbl):
        return pl.pallas_call(
            _flatten_body,
            grid=(n_cb, n_bl),
            in_specs=[pl.BlockSpec((8, 128 * _U), lambda cb, b: (cb, b))],
            out_specs=pl.BlockSpec((_U, 8, 128), lambda cb, b: (cb * n_bl + b, 0, 0)),
            out_shape=jax.ShapeDtypeStruct((n_tiles, 8, 128), jnp.float32),
        )(tbl)

    return call, n_bl * _U


@functools.lru_cache(maxsize=None)
def _make_sc_gather(V, D, B, tiles_per_cb, n_granules):
    b_per_w = B // _NUM_WORKERS
    n_ch = b_per_w // _CHUNK
    n_grp = _CHUNK // _LANES
    mesh = plsc.VectorSubcoreMesh(core_axis_name="c", subcore_axis_name="s")

    @functools.partial(
        pl.kernel,
        mesh=mesh,
        out_type=jax.ShapeDtypeStruct((D, B), jnp.float32),
        scratch_types=(
            [pltpu.VMEM((b_per_w,), jnp.int32)]
            + [pltpu.VMEM((_CHUNK,), jnp.int32)
               for _ in range(n_ch * D)]
            + [pltpu.VMEM((_CHUNK, _LANES), jnp.float32)
               for _ in range(n_ch * D)]
            + [pltpu.VMEM((D, b_per_w), jnp.float32)]
            + [pltpu.SemaphoreType.DMA for _ in range(n_ch)]
        ),
        compiler_params=pltpu.CompilerParams(use_tc_tiling_on_sc=False, needs_layout_passes=False),
    )
    def sc_gather(table_hbm, idx_hbm, out_hbm, *scratch):
        idx_v = scratch[0]
        g_bufs = scratch[1:1 + n_ch * D]
        d_bufs = scratch[1 + n_ch * D:1 + 2 * n_ch * D]
        out_v = scratch[1 + 2 * n_ch * D]
        sems = scratch[2 + 2 * n_ch * D:2 + 2 * n_ch * D + n_ch]

        wid = lax.axis_index("s") * 2 + lax.axis_index("c")
        pltpu.sync_copy(idx_hbm.at[wid], idx_v)

        iota = lax.iota(jnp.int32, _LANES)
        copies = []
        for j in range(n_ch):
            for k in range(n_grp):
                ids = idx_v[pl.ds(j * _CHUNK + k * _LANES, _LANES)]
                gbase = (lax.shift_right_logical(ids, 7) * 64
                         + jnp.bitwise_and(lax.shift_right_logical(ids, 4), 7))
                for d in range(D):
                    cb, s = d >> 3, d & 7
                    g_bufs[j * D + d][pl.ds(k * _LANES, _LANES)] = (
                        gbase + (cb * tiles_per_cb * 64 + s * 8))
            for d in range(D):
                copies.append(pltpu.async_copy(
                    table_hbm.at[g_bufs[j * D + d]], d_bufs[j * D + d], sems[j]))

        for j in range(n_ch):
            for d in range(D):
                copies[j * D + d].wait()
            for k in range(n_grp):
                ids = idx_v[pl.ds(j * _CHUNK + k * _LANES, _LANES)]
                o = jnp.bitwise_and(ids, 15)
                rows = iota + (k * _LANES)
                outcol = rows + (j * _CHUNK)
                for d in range(D):
                    v = plsc.load_gather(d_bufs[j * D + d], [rows, o])
                    plsc.store_scatter(
                        out_v, [jnp.full((_LANES,), d, jnp.int32), outcol], v)

        for d in range(D):
            pltpu.sync_copy(out_v.at[d], out_hbm.at[d, pl.ds(wid * b_per_w, b_per_w)])

    return sc_gather


def _tc_math(c_ref, d_ref, o_ref):
    C = [c_ref[k] for k in range(16)]
    d = [d_ref[k] for k in range(9)]
    a1x, a1y, a1z = d[3] + 1.0, d[4], d[5]
    a2x, a2y, a2z = d[6], d[7] + 1.0, d[8]
    n1 = lax.rsqrt(a1x * a1x + a1y * a1y + a1z * a1z)
    b1x, b1y, b1z = a1x * n1, a1y * n1, a1z * n1
    dot = b1x * a2x + b1y * a2y + b1z * a2z
    u2x, u2y, u2z = a2x - dot * b1x, a2y - dot * b1y, a2z - dot * b1z
    n2 = lax.rsqrt(u2x * u2x + u2y * u2y + u2z * u2z)
    b2x, b2y, b2z = u2x * n2, u2y * n2, u2z * n2
    b3x = b1y * b2z - b1z * b2y
    b3y = b1z * b2x - b1x * b2z
    b3z = b1x * b2y - b1y * b2x
    rot = ((b1x, b1y, b1z), (b2x, b2y, b2z), (b3x, b3y, b3z))
    dx = (d[0], d[1], d[2])
    for r in range(4):
        Cr = [C[4 * r + k] for k in range(4)]
        for cc in range(3):
            o_ref[4 * r + cc] = (
                Cr[0] * rot[0][cc] + Cr[1] * rot[1][cc] + Cr[2] * rot[2][cc]
            )
        o_ref[4 * r + 3] = Cr[0] * dx[0] + Cr[1] * dx[1] + Cr[2] * dx[2] + Cr[3]


def kernel(camtoworlds, embed_ids, embeds):
    B = camtoworlds.shape[0]
    V, D = embeds.shape
    R = B // 128
    flatten, tiles_per_cb = _make_flatten(V, D)
    flat = flatten(embeds.T)
    table = flat.reshape(flat.shape[0] * 1024 // _LANES, _LANES)
    ids = embed_ids.astype(jnp.int32).reshape(_NUM_WORKERS, B // _NUM_WORKERS)
    dT = _make_sc_gather(V, D, B, tiles_per_cb, table.shape[0])(
        table, ids).reshape(D, R, 128)
    cT = camtoworlds.reshape(B, 16).T.reshape(16, R, 128)
    oT = pl.pallas_call(
        _tc_math,
        out_shape=jax.ShapeDtypeStruct((16, R, 128), jnp.float32),
    )(cT, dT)
    return oT.reshape(16, B).T.reshape(B, 4, 4)


# speedup vs baseline: 1.4687x; 1.2144x over previous
import functools

import jax
import jax.numpy as jnp
from jax import lax
from jax.experimental import pallas as pl
from jax.experimental.pallas import tpu as pltpu
from jax.experimental.pallas import tpu_sc as plsc

_NUM_WORKERS = 32
_CHUNK = 128
_LANES = 16
_U = 64


def _flatten_body(in_ref, out_ref):
    for t in range(_U):
        out_ref[t] = in_ref[:, pl.ds(t * 128, 128)]


@functools.lru_cache(maxsize=None)
def _make_flatten(V, D):
    n_bl = (V + 128 * _U - 1) // (128 * _U)
    n_tiles = n_bl * _U

    def call(tbl):
        return pl.pallas_call(
            _flatten_body,
            grid=(n_bl,),
            in_specs=[pl.BlockSpec((8, 128 * _U), lambda b: (0, b))],
            out_specs=pl.BlockSpec((_U, 8, 128), lambda b: (b, 0, 0)),
            out_shape=jax.ShapeDtypeStruct((n_tiles, 8, 128), jnp.float32),
        )(tbl)

    return call, n_tiles


@functools.lru_cache(maxsize=None)
def _make_sc_gather(V, D, B, n_granules):
    b_per_w = B // _NUM_WORKERS
    n_ch = b_per_w // _CHUNK
    n_grp = _CHUNK // _LANES
    mesh = plsc.VectorSubcoreMesh(core_axis_name="c", subcore_axis_name="s")

    @functools.partial(
        pl.kernel,
        mesh=mesh,
        out_type=jax.ShapeDtypeStruct((D, B), jnp.float32),
        scratch_types=(
            [pltpu.VMEM((b_per_w,), jnp.int32)]
            + [pltpu.VMEM((_CHUNK,), jnp.int32)
               for _ in range(n_ch * D)]
            + [pltpu.VMEM((_CHUNK, _LANES), jnp.float32)
               for _ in range(n_ch * D)]
            + [pltpu.VMEM((D, b_per_w), jnp.float32)]
            + [pltpu.SemaphoreType.DMA for _ in range(n_ch)]
        ),
        compiler_params=pltpu.CompilerParams(use_tc_tiling_on_sc=False, needs_layout_passes=False),
    )
    def sc_gather(table_hbm, table8_hbm, idx_hbm, out_hbm, *scratch):
        idx_v = scratch[0]
        g_bufs = scratch[1:1 + n_ch * D]
        d_bufs = scratch[1 + n_ch * D:1 + 2 * n_ch * D]
        out_v = scratch[1 + 2 * n_ch * D]
        sems = scratch[2 + 2 * n_ch * D:2 + 2 * n_ch * D + n_ch]

        wid = lax.axis_index("s") * 2 + lax.axis_index("c")
        pltpu.sync_copy(idx_hbm.at[wid], idx_v)

        iota = lax.iota(jnp.int32, _LANES)
        copies = []
        for j in range(n_ch):
            for k in range(n_grp):
                ids = idx_v[pl.ds(j * _CHUNK + k * _LANES, _LANES)]
                g4 = lax.shift_right_logical(ids, 4)
                gbase = (lax.shift_right_logical(ids, 7) * 64
                         + jnp.bitwise_and(g4, 7))
                for d in range(8):
                    g_bufs[j * D + d][pl.ds(k * _LANES, _LANES)] = gbase + d * 8
                g_bufs[j * D + 8][pl.ds(k * _LANES, _LANES)] = g4
            for d in range(8):
                copies.append(pltpu.async_copy(
                    table_hbm.at[g_bufs[j * D + d]], d_bufs[j * D + d], sems[j]))
            copies.append(pltpu.async_copy(
                table8_hbm.at[g_bufs[j * D + 8]], d_bufs[j * D + 8], sems[j]))

        for j in range(n_ch):
            for d in range(D):
                copies[j * D + d].wait()
            for k in range(n_grp):
                ids = idx_v[pl.ds(j * _CHUNK + k * _LANES, _LANES)]
                o = jnp.bitwise_and(ids, 15)
                rows = iota + (k * _LANES)
                outcol = rows + (j * _CHUNK)
                for d in range(D):
                    v = plsc.load_gather(d_bufs[j * D + d], [rows, o])
                    plsc.store_scatter(
                        out_v, [jnp.full((_LANES,), d, jnp.int32), outcol], v)

        for d in range(D):
            pltpu.sync_copy(out_v.at[d], out_hbm.at[d, pl.ds(wid * b_per_w, b_per_w)])

    return sc_gather


def _tc_math(c_ref, d_ref, o_ref):
    C = [c_ref[k] for k in range(16)]
    d = [d_ref[k] for k in range(9)]
    a1x, a1y, a1z = d[3] + 1.0, d[4], d[5]
    a2x, a2y, a2z = d[6], d[7] + 1.0, d[8]
    n1 = lax.rsqrt(a1x * a1x + a1y * a1y + a1z * a1z)
    b1x, b1y, b1z = a1x * n1, a1y * n1, a1z * n1
    dot = b1x * a2x + b1y * a2y + b1z * a2z
    u2x, u2y, u2z = a2x - dot * b1x, a2y - dot * b1y, a2z - dot * b1z
    n2 = lax.rsqrt(u2x * u2x + u2y * u2y + u2z * u2z)
    b2x, b2y, b2z = u2x * n2, u2y * n2, u2z * n2
    b3x = b1y * b2z - b1z * b2y
    b3y = b1z * b2x - b1x * b2z
    b3z = b1x * b2y - b1y * b2x
    rot = ((b1x, b1y, b1z), (b2x, b2y, b2z), (b3x, b3y, b3z))
    dx = (d[0], d[1], d[2])
    for r in range(4):
        Cr = [C[4 * r + k] for k in range(4)]
        for cc in range(3):
            o_ref[4 * r + cc] = (
                Cr[0] * rot[0][cc] + Cr[1] * rot[1][cc] + Cr[2] * rot[2][cc]
            )
        o_ref[4 * r + 3] = Cr[0] * dx[0] + Cr[1] * dx[1] + Cr[2] * dx[2] + Cr[3]


def kernel(camtoworlds, embed_ids, embeds):
    B = camtoworlds.shape[0]
    V, D = embeds.shape
    R = B // 128
    flatten, _ = _make_flatten(V, D)
    tblT = embeds.T
    flat = flatten(tblT)
    table = flat.reshape(flat.shape[0] * 1024 // _LANES, _LANES)
    table8 = tblT[8].reshape(V // _LANES, _LANES)
    ids = embed_ids.astype(jnp.int32).reshape(_NUM_WORKERS, B // _NUM_WORKERS)
    dT = _make_sc_gather(V, D, B, table.shape[0])(
        table, table8, ids).reshape(D, R, 128)
    cT = camtoworlds.reshape(B, 16).T.reshape(16, R, 128)
    oT = pl.pallas_call(
        _tc_math,
        out_shape=jax.ShapeDtypeStruct((16, R, 128), jnp.float32),
    )(cT, dT)
    return oT.reshape(16, B).T.reshape(B, 4, 4)


# speedup vs baseline: 2.0552x; 1.3993x over previous
import functools

import jax
import jax.numpy as jnp
from jax import lax
from jax.experimental import pallas as pl
from jax.experimental.pallas import tpu as pltpu
from jax.experimental.pallas import tpu_sc as plsc

_NUM_WORKERS = 32
_CHUNK = 128
_LANES = 16
_U = 256


def _flatten_body(in_ref, out_ref):
    for t in range(_U):
        out_ref[t] = in_ref[:, pl.ds(t * 128, 128)]


@functools.lru_cache(maxsize=None)
def _make_flatten(V, D):
    n_bl = (V + 128 * _U - 1) // (128 * _U)
    n_tiles = n_bl * _U

    def call(tbl):
        return pl.pallas_call(
            _flatten_body,
            grid=(n_bl,),
            in_specs=[pl.BlockSpec((8, 128 * _U), lambda b: (0, b))],
            out_specs=pl.BlockSpec((_U, 8, 128), lambda b: (b, 0, 0)),
            out_shape=jax.ShapeDtypeStruct((n_tiles, 8, 128), jnp.float32),
        )(tbl)

    return call, n_tiles


@functools.lru_cache(maxsize=None)
def _make_sc_gather(V, D, B, n_granules):
    b_per_w = B // _NUM_WORKERS
    n_ch = b_per_w // _CHUNK
    n_grp = _CHUNK // _LANES
    mesh = plsc.VectorSubcoreMesh(core_axis_name="c", subcore_axis_name="s")

    @functools.partial(
        pl.kernel,
        mesh=mesh,
        out_type=jax.ShapeDtypeStruct((D, B), jnp.float32),
        scratch_types=(
            [pltpu.VMEM((b_per_w,), jnp.int32)]
            + [pltpu.VMEM((_CHUNK,), jnp.int32)
               for _ in range(n_ch * D)]
            + [pltpu.VMEM((_CHUNK, _LANES), jnp.float32)
               for _ in range(n_ch * D)]
            + [pltpu.VMEM((D, b_per_w), jnp.float32)]
            + [pltpu.SemaphoreType.DMA for _ in range(n_ch)]
        ),
        compiler_params=pltpu.CompilerParams(use_tc_tiling_on_sc=False, needs_layout_passes=False),
    )
    def sc_gather(table_hbm, table8_hbm, idx_hbm, out_hbm, *scratch):
        idx_v = scratch[0]
        g_bufs = scratch[1:1 + n_ch * D]
        d_bufs = scratch[1 + n_ch * D:1 + 2 * n_ch * D]
        out_v = scratch[1 + 2 * n_ch * D]
        sems = scratch[2 + 2 * n_ch * D:2 + 2 * n_ch * D + n_ch]

        wid = lax.axis_index("s") * 2 + lax.axis_index("c")
        pltpu.sync_copy(idx_hbm.at[wid], idx_v)

        iota = lax.iota(jnp.int32, _LANES)
        copies = []
        for j in range(n_ch):
            for k in range(n_grp):
                ids = idx_v[pl.ds(j * _CHUNK + k * _LANES, _LANES)]
                g4 = lax.shift_right_logical(ids, 4)
                gbase = (lax.shift_right_logical(ids, 7) * 64
                         + jnp.bitwise_and(g4, 7))
                for d in range(8):
                    g_bufs[j * D + d][pl.ds(k * _LANES, _LANES)] = gbase + d * 8
                g_bufs[j * D + 8][pl.ds(k * _LANES, _LANES)] = g4
            for d in range(8):
                copies.append(pltpu.async_copy(
                    table_hbm.at[g_bufs[j * D + d]], d_bufs[j * D + d], sems[j]))
            copies.append(pltpu.async_copy(
                table8_hbm.at[g_bufs[j * D + 8]], d_bufs[j * D + 8], sems[j]))

        for j in range(n_ch):
            for d in range(D):
                copies[j * D + d].wait()
            for k in range(n_grp):
                ids = idx_v[pl.ds(j * _CHUNK + k * _LANES, _LANES)]
                o = jnp.bitwise_and(ids, 15)
                rows = iota + (k * _LANES)
                outcol = rows + (j * _CHUNK)
                for d in range(D):
                    v = plsc.load_gather(d_bufs[j * D + d], [rows, o])
                    plsc.store_scatter(
                        out_v, [jnp.full((_LANES,), d, jnp.int32), outcol], v)

        for d in range(D):
            pltpu.sync_copy(out_v.at[d], out_hbm.at[d, pl.ds(wid * b_per_w, b_per_w)])

    return sc_gather


def _tc_math(c_ref, d_ref, o_ref):
    C = [c_ref[k] for k in range(16)]
    d = [d_ref[k] for k in range(9)]
    a1x, a1y, a1z = d[3] + 1.0, d[4], d[5]
    a2x, a2y, a2z = d[6], d[7] + 1.0, d[8]
    n1 = lax.rsqrt(a1x * a1x + a1y * a1y + a1z * a1z)
    b1x, b1y, b1z = a1x * n1, a1y * n1, a1z * n1
    dot = b1x * a2x + b1y * a2y + b1z * a2z
    u2x, u2y, u2z = a2x - dot * b1x, a2y - dot * b1y, a2z - dot * b1z
    n2 = lax.rsqrt(u2x * u2x + u2y * u2y + u2z * u2z)
    b2x, b2y, b2z = u2x * n2, u2y * n2, u2z * n2
    b3x = b1y * b2z - b1z * b2y
    b3y = b1z * b2x - b1x * b2z
    b3z = b1x * b2y - b1y * b2x
    rot = ((b1x, b1y, b1z), (b2x, b2y, b2z), (b3x, b3y, b3z))
    dx = (d[0], d[1], d[2])
    for r in range(4):
        Cr = [C[4 * r + k] for k in range(4)]
        for cc in range(3):
            o_ref[4 * r + cc] = (
                Cr[0] * rot[0][cc] + Cr[1] * rot[1][cc] + Cr[2] * rot[2][cc]
            )
        o_ref[4 * r + 3] = Cr[0] * dx[0] + Cr[1] * dx[1] + Cr[2] * dx[2] + Cr[3]


def kernel(camtoworlds, embed_ids, embeds):
    B = camtoworlds.shape[0]
    V, D = embeds.shape
    R = B // 128
    flatten, _ = _make_flatten(V, D)
    tblT = embeds.T
    flat = flatten(tblT)
    table = flat.reshape(flat.shape[0] * 1024 // _LANES, _LANES)
    table8 = tblT[8].reshape(V // _LANES, _LANES)
    ids = embed_ids.astype(jnp.int32).reshape(_NUM_WORKERS, B // _NUM_WORKERS)
    dT = _make_sc_gather(V, D, B, table.shape[0])(
        table, table8, ids).reshape(D, R, 128)
    cT = camtoworlds.reshape(B, 16).T.reshape(16, R, 128)
    oT = pl.pallas_call(
        _tc_math,
        out_shape=jax.ShapeDtypeStruct((16, R, 128), jnp.float32),
    )(cT, dT)
    return oT.reshape(16, B).T.reshape(B, 4, 4)


# speedup vs baseline: 2.2443x; 1.0920x over previous
import functools

import jax
import jax.numpy as jnp
from jax import lax
from jax.experimental import pallas as pl
from jax.experimental.pallas import tpu as pltpu
from jax.experimental.pallas import tpu_sc as plsc

_NUM_WORKERS = 32
_CHUNK = 128
_LANES = 16
_U = 512


def _flatten_body(in_ref, out_ref):
    for t in range(_U):
        out_ref[t] = in_ref[:, pl.ds(t * 128, 128)]


@functools.lru_cache(maxsize=None)
def _make_flatten(V, D):
    n_bl = (V + 128 * _U - 1) // (128 * _U)
    n_tiles = n_bl * _U

    def call(tbl):
        return pl.pallas_call(
            _flatten_body,
            grid=(n_bl,),
            in_specs=[pl.BlockSpec((8, 128 * _U), lambda b: (0, b))],
            out_specs=pl.BlockSpec((_U, 8, 128), lambda b: (b, 0, 0)),
            out_shape=jax.ShapeDtypeStruct((n_tiles, 8, 128), jnp.float32),
        )(tbl)

    return call, n_tiles


@functools.lru_cache(maxsize=None)
def _make_sc_gather(V, D, B, n_granules):
    b_per_w = B // _NUM_WORKERS
    n_ch = b_per_w // _CHUNK
    n_grp = _CHUNK // _LANES
    mesh = plsc.VectorSubcoreMesh(core_axis_name="c", subcore_axis_name="s")

    @functools.partial(
        pl.kernel,
        mesh=mesh,
        out_type=jax.ShapeDtypeStruct((D, B), jnp.float32),
        scratch_types=(
            [pltpu.VMEM((b_per_w,), jnp.int32)]
            + [pltpu.VMEM((_CHUNK,), jnp.int32)
               for _ in range(n_ch * D)]
            + [pltpu.VMEM((_CHUNK, _LANES), jnp.float32)
               for _ in range(n_ch * D)]
            + [pltpu.VMEM((D, b_per_w), jnp.float32)]
            + [pltpu.SemaphoreType.DMA for _ in range(n_ch)]
        ),
        compiler_params=pltpu.CompilerParams(use_tc_tiling_on_sc=False, needs_layout_passes=False),
    )
    def sc_gather(table_hbm, table8_hbm, idx_hbm, out_hbm, *scratch):
        idx_v = scratch[0]
        g_bufs = scratch[1:1 + n_ch * D]
        d_bufs = scratch[1 + n_ch * D:1 + 2 * n_ch * D]
        out_v = scratch[1 + 2 * n_ch * D]
        sems = scratch[2 + 2 * n_ch * D:2 + 2 * n_ch * D + n_ch]

        wid = lax.axis_index("s") * 2 + lax.axis_index("c")
        pltpu.sync_copy(idx_hbm.at[wid], idx_v)

        iota = lax.iota(jnp.int32, _LANES)
        copies = []
        for j in range(n_ch):
            for k in range(n_grp):
                ids = idx_v[pl.ds(j * _CHUNK + k * _LANES, _LANES)]
                g4 = lax.shift_right_logical(ids, 4)
                gbase = (lax.shift_right_logical(ids, 7) * 64
                         + jnp.bitwise_and(g4, 7))
                for d in range(8):
                    g_bufs[j * D + d][pl.ds(k * _LANES, _LANES)] = gbase + d * 8
                g_bufs[j * D + 8][pl.ds(k * _LANES, _LANES)] = g4
            for d in range(8):
                copies.append(pltpu.async_copy(
                    table_hbm.at[g_bufs[j * D + d]], d_bufs[j * D + d], sems[j]))
            copies.append(pltpu.async_copy(
                table8_hbm.at[g_bufs[j * D + 8]], d_bufs[j * D + 8], sems[j]))

        for j in range(n_ch):
            for d in range(D):
                copies[j * D + d].wait()
            for k in range(n_grp):
                ids = idx_v[pl.ds(j * _CHUNK + k * _LANES, _LANES)]
                o = jnp.bitwise_and(ids, 15)
                rows = iota + (k * _LANES)
                outcol = rows + (j * _CHUNK)
                for d in range(D):
                    v = plsc.load_gather(d_bufs[j * D + d], [rows, o])
                    plsc.store_scatter(
                        out_v, [jnp.full((_LANES,), d, jnp.int32), outcol], v)

        for d in range(D):
            pltpu.sync_copy(out_v.at[d], out_hbm.at[d, pl.ds(wid * b_per_w, b_per_w)])

    return sc_gather


def _tc_math(c_ref, d_ref, o_ref):
    C = [c_ref[k] for k in range(16)]
    d = [d_ref[k] for k in range(9)]
    a1x, a1y, a1z = d[3] + 1.0, d[4], d[5]
    a2x, a2y, a2z = d[6], d[7] + 1.0, d[8]
    n1 = lax.rsqrt(a1x * a1x + a1y * a1y + a1z * a1z)
    b1x, b1y, b1z = a1x * n1, a1y * n1, a1z * n1
    dot = b1x * a2x + b1y * a2y + b1z * a2z
    u2x, u2y, u2z = a2x - dot * b1x, a2y - dot * b1y, a2z - dot * b1z
    n2 = lax.rsqrt(u2x * u2x + u2y * u2y + u2z * u2z)
    b2x, b2y, b2z = u2x * n2, u2y * n2, u2z * n2
    b3x = b1y * b2z - b1z * b2y
    b3y = b1z * b2x - b1x * b2z
    b3z = b1x * b2y - b1y * b2x
    rot = ((b1x, b1y, b1z), (b2x, b2y, b2z), (b3x, b3y, b3z))
    dx = (d[0], d[1], d[2])
    for r in range(4):
        Cr = [C[4 * r + k] for k in range(4)]
        for cc in range(3):
            o_ref[4 * r + cc] = (
                Cr[0] * rot[0][cc] + Cr[1] * rot[1][cc] + Cr[2] * rot[2][cc]
            )
        o_ref[4 * r + 3] = Cr[0] * dx[0] + Cr[1] * dx[1] + Cr[2] * dx[2] + Cr[3]


def kernel(camtoworlds, embed_ids, embeds):
    B = camtoworlds.shape[0]
    V, D = embeds.shape
    R = B // 128
    flatten, _ = _make_flatten(V, D)
    tblT = embeds.T
    flat = flatten(tblT)
    table = flat.reshape(flat.shape[0] * 1024 // _LANES, _LANES)
    table8 = tblT[8].reshape(V // _LANES, _LANES)
    ids = embed_ids.astype(jnp.int32).reshape(_NUM_WORKERS, B // _NUM_WORKERS)
    dT = _make_sc_gather(V, D, B, table.shape[0])(
        table, table8, ids).reshape(D, R, 128)
    cT = camtoworlds.reshape(B, 16).T.reshape(16, R, 128)
    oT = pl.pallas_call(
        _tc_math,
        out_shape=jax.ShapeDtypeStruct((16, R, 128), jnp.float32),
    )(cT, dT)
    return oT.reshape(16, B).T.reshape(B, 4, 4)
